# baseline (speedup 1.0000x reference)
import jax
import jax.numpy as jnp
from jax.experimental import pallas as pl

_BR = 128
_NQ = 4


def _embed_body(ts_ref, nv_ref, mk_ref, wdr_ref, wvr_ref, bdr_ref, bvr_ref,
                tbr_ref, out_ref):
    Q = out_ref.shape[1] // _NQ
    reps = Q // ts_ref.shape[1]
    step = 8
    for c in range(0, _BR, step):
        rows = slice(c, c + step)
        ts16 = ts_ref[rows, :]
        nv16 = nv_ref[rows, :]
        mk16 = mk_ref[rows, :]
        for _ in range(reps.bit_length() - 1):
            ts16 = jnp.concatenate([ts16, ts16], axis=1)
            nv16 = jnp.concatenate([nv16, nv16], axis=1)
            mk16 = jnp.concatenate([mk16, mk16], axis=1)
        for q in range(_NQ):
            sl = pl.ds(q * Q, Q)
            bias = bdr_ref[:, sl] + bvr_ref[:, sl] + tbr_ref[:, sl]
            out_ref[rows, sl] = (ts16 * wdr_ref[:, sl] + nv16 * wvr_ref[:, sl]
                                 + bias) * mk16


def kernel(timestamp, numerical_value, mask, code, W_date, b_date, table,
           W_val, b_val):
    B, L = timestamp.shape
    D = W_date.shape[0]
    del code

    wdr = jnp.repeat(W_date[:, 0], L).reshape(1, D * L)
    wvr = jnp.repeat(W_val[:, 0], L).reshape(1, D * L)
    bdr = jnp.repeat(b_date, L).reshape(1, D * L)
    bvr = jnp.repeat(b_val, L).reshape(1, D * L)
    tbr = jnp.repeat(table[0, :], L).reshape(1, D * L)

    row_spec = pl.BlockSpec((_BR, L), lambda i: (i, 0))
    pat_spec = pl.BlockSpec((1, D * L), lambda i: (0, 0))

    out2 = pl.pallas_call(
        _embed_body,
        grid=(B // _BR,),
        in_specs=[row_spec, row_spec, row_spec,
                  pat_spec, pat_spec, pat_spec, pat_spec, pat_spec],
        out_specs=pl.BlockSpec((_BR, D * L), lambda i: (i, 0)),
        out_shape=jax.ShapeDtypeStruct((B, D * L), jnp.float32),
    )(timestamp, numerical_value, mask.reshape(B, L),
      wdr, wvr, bdr, bvr, tbr)
    return out2.reshape(B, D, L)


# speedup vs baseline: 1.1528x; 1.1528x over previous
import jax
import jax.numpy as jnp
from jax.experimental import pallas as pl

_BR = 128
_NQ = 4


def _embed_body(ts_ref, nv_ref, mk_ref, wdr_ref, wvr_ref, bdr_ref, bvr_ref,
                tbr_ref, out_ref):
    Q = out_ref.shape[1] // _NQ
    reps = Q // ts_ref.shape[1]
    qsl = [pl.ds(q * Q, Q) for q in range(_NQ)]
    biasq = [bdr_ref[:, sl] + bvr_ref[:, sl] + tbr_ref[:, sl] for sl in qsl]
    step = 8
    for c in range(0, _BR, step):
        rows = slice(c, c + step)
        ts16 = jnp.concatenate([ts_ref[rows, :]] * reps, axis=1)
        nv16 = jnp.concatenate([nv_ref[rows, :]] * reps, axis=1)
        mk16 = jnp.concatenate([mk_ref[rows, :]] * reps, axis=1)
        for q in range(_NQ):
            sl = qsl[q]
            out_ref[rows, sl] = (ts16 * wdr_ref[:, sl] + nv16 * wvr_ref[:, sl]
                                 + biasq[q]) * mk16


def kernel(timestamp, numerical_value, mask, code, W_date, b_date, table,
           W_val, b_val):
    B, L = timestamp.shape
    D = W_date.shape[0]
    del code

    wdr = jnp.repeat(W_date[:, 0], L).reshape(1, D * L)
    wvr = jnp.repeat(W_val[:, 0], L).reshape(1, D * L)
    bdr = jnp.repeat(b_date, L).reshape(1, D * L)
    bvr = jnp.repeat(b_val, L).reshape(1, D * L)
    tbr = jnp.repeat(table[0, :], L).reshape(1, D * L)

    row_spec = pl.BlockSpec((_BR, L), lambda i: (i, 0))
    pat_spec = pl.BlockSpec((1, D * L), lambda i: (0, 0))

    out2 = pl.pallas_call(
        _embed_body,
        grid=(B // _BR,),
        in_specs=[row_spec, row_spec, row_spec,
                  pat_spec, pat_spec, pat_spec, pat_spec, pat_spec],
        out_specs=pl.BlockSpec((_BR, D * L), lambda i: (i, 0)),
        out_shape=jax.ShapeDtypeStruct((B, D * L), jnp.float32),
    )(timestamp, numerical_value, mask.reshape(B, L),
      wdr, wvr, bdr, bvr, tbr)
    return out2.reshape(B, D, L)
